# Initial kernel scaffold; baseline (speedup 1.0000x reference)
import jax
import jax.numpy as jnp
from jax.experimental import pallas as pl


def kernel(x, edge_index, batch, W1, b1, W2, b2, Wfc, bfc):
    raise NotImplementedError("write your pallas kernel here")


# speedup vs baseline: 13.3382x; 13.3382x over previous
import functools

import jax
import jax.numpy as jnp
from jax import lax
from jax.experimental import pallas as pl
from jax.experimental.pallas import tpu as pltpu
from jax.experimental.pallas import tpu_sc as plsc

NC = 2
NS = 16
NW = NC * NS
CHUNK = 128
G = 64


def _make_deg_kernel(nch, drows, dpt):
    mesh = plsc.VectorSubcoreMesh(core_axis_name="c", subcore_axis_name="s",
                                  num_cores=NC, num_subcores=NS)

    @functools.partial(
        pl.kernel,
        out_type=jax.ShapeDtypeStruct((NC, drows), jnp.float32),
        mesh=mesh,
        scratch_types=[
            pltpu.VMEM((nch, CHUNK), jnp.int32),
            pltpu.VMEM((CHUNK,), jnp.float32),
            pltpu.VMEM((dpt,), jnp.float32),
            pltpu.VMEM_SHARED((drows,), jnp.float32),
        ],
    )
    def deg_kernel(dst_hbm, degp_hbm, dst_v, ones_v, zbuf, deg_sh):
        c = lax.axis_index("c")
        s = lax.axis_index("s")
        wid = s * NC + c

        def fill_ones(i, _):
            ones_v[pl.ds(i * 16, 16)] = jnp.ones((16,), jnp.float32)
            return 0

        lax.fori_loop(0, CHUNK // 16, fill_ones, 0)

        def fill_zero(i, _):
            zbuf[pl.ds(i * 16, 16)] = jnp.zeros((16,), jnp.float32)
            return 0

        lax.fori_loop(0, dpt // 16, fill_zero, 0)
        pltpu.sync_copy(zbuf, deg_sh.at[pl.ds(s * dpt, dpt)])
        plsc.subcore_barrier()

        pltpu.sync_copy(dst_hbm.at[wid], dst_v)

        def chunk(j, _):
            pltpu.sync_copy(ones_v, deg_sh.at[dst_v.at[j]], add=True)
            return 0

        lax.fori_loop(0, nch, chunk, 0)
        plsc.subcore_barrier()
        pltpu.sync_copy(deg_sh.at[pl.ds(s * dpt, dpt)],
                        degp_hbm.at[c, pl.ds(s * dpt, dpt)])

    return deg_kernel


def _make_mp_kernel(nch, zrows, rpt, f):
    mesh = plsc.VectorSubcoreMesh(core_axis_name="c", subcore_axis_name="s",
                                  num_cores=NC, num_subcores=NS)

    @functools.partial(
        pl.kernel,
        out_type=jax.ShapeDtypeStruct((NC, zrows, f), jnp.float32),
        mesh=mesh,
        scratch_types=[
            pltpu.VMEM((nch, CHUNK), jnp.int32),
            pltpu.VMEM((nch, CHUNK), jnp.int32),
            pltpu.VMEM((CHUNK, f), jnp.float32),
            pltpu.VMEM_SHARED((zrows, f), jnp.float32),
            pltpu.SemaphoreType.DMA,
        ],
    )
    def mp_kernel(y_hbm, src_hbm, dst_hbm, z_hbm, src_v, dst_v, rows_v, z_sh, sem):
        c = lax.axis_index("c")
        s = lax.axis_index("s")
        wid = s * NC + c

        pltpu.sync_copy(y_hbm.at[pl.ds(s * rpt, rpt)],
                        z_sh.at[pl.ds(s * rpt, rpt)])
        plsc.subcore_barrier()

        pltpu.sync_copy(src_hbm.at[wid], src_v)
        pltpu.sync_copy(dst_hbm.at[wid], dst_v)

        def chunk(j, _):
            pltpu.async_copy(y_hbm.at[src_v.at[j]], rows_v, sem).wait()
            pltpu.sync_copy(rows_v, z_sh.at[dst_v.at[j]], add=True)
            return 0

        lax.fori_loop(0, nch, chunk, 0)
        plsc.subcore_barrier()
        pltpu.sync_copy(z_sh.at[pl.ds(s * rpt, rpt)],
                        z_hbm.at[c, pl.ds(s * rpt, rpt)])

    return mp_kernel


def _t1_body(x_ref, w_ref, d0_ref, d1_ref, y_ref, dinv_ref):
    dinv = lax.rsqrt(d0_ref[...] + d1_ref[...] + 1.0)
    y_ref[...] = dinv * jnp.dot(x_ref[...], w_ref[...],
                                preferred_element_type=jnp.float32)
    dinv_ref[...] = dinv


def _t2_body(z0_ref, z1_ref, y_ref, dinv_ref, b_ref, w_ref, out_ref):
    dinv = dinv_ref[...]
    h = dinv * (z0_ref[...] + z1_ref[...] - y_ref[...]) + b_ref[...]
    h = jnp.maximum(h, 0.0)
    out_ref[...] = dinv * jnp.dot(h, w_ref[...],
                                  preferred_element_type=jnp.float32)


def _t3_body(z0_ref, z1_ref, y_ref, dinv_ref, b_ref, batch_ref, wfc_ref,
             bfc_ref, out_ref, s_acc, c_acc, *, blk):
    i = pl.program_id(0)

    @pl.when(i == 0)
    def _():
        s_acc[...] = jnp.zeros_like(s_acc)
        c_acc[...] = jnp.zeros_like(c_acc)

    h = dinv_ref[...] * (z0_ref[...] + z1_ref[...] - y_ref[...]) + b_ref[...]
    h = jnp.maximum(h, 0.0)
    seg = lax.broadcasted_iota(jnp.int32, (blk, G), 1)
    oh = (batch_ref[...] == seg).astype(jnp.float32)
    dn = (((0,), (0,)), ((), ()))
    s_acc[...] += lax.dot_general(oh, h, dn,
                                  preferred_element_type=jnp.float32)
    c_acc[...] += lax.dot_general(oh, jnp.ones((blk, 8), jnp.float32), dn,
                                  preferred_element_type=jnp.float32)

    @pl.when(i == pl.num_programs(0) - 1)
    def _():
        cnt = jnp.maximum(lax.slice(c_acc[...], (0, 0), (G, 1)), 1.0)
        pooled = s_acc[...] / cnt
        out_ref[...] = jnp.dot(pooled, wfc_ref[...],
                               preferred_element_type=jnp.float32) + bfc_ref[...]


def _row_spec(blk, f):
    return pl.BlockSpec((blk, f), lambda i: (i, 0))


def _full_spec(shape):
    return pl.BlockSpec(shape, lambda i: tuple(0 for _ in shape))


def kernel(x, edge_index, batch, W1, b1, W2, b2, Wfc, bfc):
    n, f = x.shape
    h = W1.shape[1]
    e = edge_index.shape[1]

    src = edge_index[0].astype(jnp.int32)
    dst = edge_index[1].astype(jnp.int32)
    batch32 = batch.astype(jnp.int32).reshape(n, 1)

    nch = -(-e // (NW * CHUNK))
    ep = NW * nch * CHUNK
    src_p = jnp.concatenate(
        [src, jnp.zeros((ep - e,), jnp.int32)]).reshape(NW, nch, CHUNK)
    dst_p = jnp.concatenate(
        [dst, jnp.full((ep - e,), n, jnp.int32)]).reshape(NW, nch, CHUNK)

    zrows = ((n + 1 + NS * 8 - 1) // (NS * 8)) * (NS * 8)
    rpt = zrows // NS
    drows = ((n + 1 + NS * 16 - 1) // (NS * 16)) * (NS * 16)
    dpt = drows // NS

    deg_kernel = _make_deg_kernel(nch, drows, dpt)
    mp_kernel = _make_mp_kernel(nch, zrows, rpt, f)

    degp = deg_kernel(dst_p)
    d0 = degp[0, :n].reshape(n, 1)
    d1 = degp[1, :n].reshape(n, 1)

    blk = 1000
    nb = n // blk

    t1 = pl.pallas_call(
        _t1_body,
        grid=(nb,),
        in_specs=[_row_spec(blk, f), _full_spec((f, h)),
                  _row_spec(blk, 1), _row_spec(blk, 1)],
        out_specs=[_row_spec(blk, h), _row_spec(blk, 1)],
        out_shape=[jax.ShapeDtypeStruct((n, h), jnp.float32),
                   jax.ShapeDtypeStruct((n, 1), jnp.float32)],
    )
    y1, dinv = t1(x, W1, d0, d1)

    pad = jnp.zeros((zrows - n, h), jnp.float32)
    zp1 = mp_kernel(jnp.concatenate([y1, pad]), src_p, dst_p)

    t2 = pl.pallas_call(
        _t2_body,
        grid=(nb,),
        in_specs=[_row_spec(blk, h), _row_spec(blk, h), _row_spec(blk, h),
                  _row_spec(blk, 1), _full_spec((1, h)), _full_spec((h, h))],
        out_specs=_row_spec(blk, h),
        out_shape=jax.ShapeDtypeStruct((n, h), jnp.float32),
    )
    y2 = t2(zp1[0, :n], zp1[1, :n], y1, dinv, b1.reshape(1, h), W2)

    zp2 = mp_kernel(jnp.concatenate([y2, pad]), src_p, dst_p)

    t3 = pl.pallas_call(
        functools.partial(_t3_body, blk=blk),
        grid=(nb,),
        in_specs=[_row_spec(blk, h), _row_spec(blk, h), _row_spec(blk, h),
                  _row_spec(blk, 1), _full_spec((1, h)),
                  pl.BlockSpec((blk, 1), lambda i: (i, 0)),
                  _full_spec((h, 1)), _full_spec((1, 1))],
        out_specs=_full_spec((G, 1)),
        out_shape=jax.ShapeDtypeStruct((G, 1), jnp.float32),
        scratch_shapes=[pltpu.VMEM((G, h), jnp.float32),
                        pltpu.VMEM((G, 8), jnp.float32)],
        compiler_params=pltpu.CompilerParams(
            dimension_semantics=("arbitrary",)),
    )
    return t3(zp2[0, :n], zp2[1, :n], y2, dinv, b2.reshape(1, h),
              batch32, Wfc, bfc.reshape(1, 1))
